# baseline (speedup 1.0000x reference)
import functools

import jax
import jax.numpy as jnp
from jax import lax
from jax.experimental import pallas as pl
from jax.experimental.pallas import tpu as pltpu
from jax.experimental.pallas import tpu_sc as plsc

_N = 10000
_E = 320000
_G = 64
_NP = 10240
_NC = 2
_NS = 16
_NW = _NC * _NS
_EP = 327680
_CH = 128
_NCHE = _EP // (_NW * _CH)
_NCHF = _EP // (_NS * _CH)
_RPT = _NP // _NS

_SC_PARAMS = pltpu.CompilerParams(use_tc_tiling_on_sc=False)


def _ring_pipeline(nch, idx_start, idx_wait, gather_start, gather_wait,
                   scatter_start, scatter_wait):
  for c in range(8):
    idx_start(c, c)
  for c in range(3):
    idx_wait(c)
    gather_start(c, c, c, c % 2)

  def body(j, carry):
    c0 = j * 8
    for t in range(8):
      c = c0 + t
      cn = c + 3
      ci = c + 7
      bn = (t + 3) % 4
      sn = (t + 3) % 8
      bi = (t + 7) % 8

      @pl.when(c >= 1)
      def _():
        scatter_wait((t + 3) % 4)

      @pl.when(ci < nch)
      def _():
        idx_start(ci, bi)

      @pl.when(cn < nch)
      def _():
        idx_wait(sn)
        gather_start(cn, bn, sn, (t + 3) % 2)

      gather_wait(t % 4)
      scatter_start(c, t % 4, t % 8)
    return carry

  lax.fori_loop(0, nch // 8, body, 0)
  scatter_wait((nch - 1) % 4)


def _ring_pipeline_preidx(nch, gather_start, gather_wait, scatter_start,
                          scatter_wait):
  for b in range(3):
    gather_start(b, b)

  def body(j, carry):
    c0 = j * 4
    for t in range(4):
      c = c0 + t
      cn = c + 3
      bn = (t + 3) % 4

      @pl.when(cn < nch)
      def _():
        @pl.when(c >= 1)
        def _():
          scatter_wait(bn)
        gather_start(cn, bn)

      gather_wait(t)
      scatter_start(c, t)
    return carry

  lax.fori_loop(0, nch // 4, body, 0)
  for b in range(4):
    scatter_wait(b)


def _make_prop_feat():
  mesh = plsc.VectorSubcoreMesh(core_axis_name="c", subcore_axis_name="s")

  @functools.partial(
      pl.kernel,
      out_type=jax.ShapeDtypeStruct((_NC, _NP, 64), jnp.float32),
      mesh=mesh,
      compiler_params=_SC_PARAMS,
      scratch_types=[
          pltpu.VMEM((8, 2, _CH), jnp.int32),
          pltpu.VMEM((_CH, 64), jnp.float32),
          pltpu.VMEM((_CH, 64), jnp.float32),
          pltpu.VMEM((_CH, 64), jnp.float32),
          pltpu.VMEM((_CH, 64), jnp.float32),
          pltpu.VMEM_SHARED((_NP, 64), jnp.float32),
          pltpu.VMEM_SHARED((_NP, 64), jnp.float32),
          pltpu.SemaphoreType.DMA,
          pltpu.SemaphoreType.DMA,
          pltpu.SemaphoreType.DMA,
          pltpu.SemaphoreType.DMA,
          pltpu.SemaphoreType.DMA,
          pltpu.SemaphoreType.DMA,
          pltpu.SemaphoreType.DMA,
          pltpu.SemaphoreType.DMA,
          pltpu.SemaphoreType.DMA,
          pltpu.SemaphoreType.DMA,
          pltpu.SemaphoreType.DMA,
          pltpu.SemaphoreType.DMA,
          pltpu.SemaphoreType.DMA,
          pltpu.SemaphoreType.DMA,
          pltpu.SemaphoreType.DMA,
          pltpu.SemaphoreType.DMA,
      ],
  )
  def prop(g_hbm, rc_hbm, zero_hbm, out_hbm,
           idx_v, b0, b1, b2, b3, acc_sh, src_sh,
           q0, q1, q2, q3, s0, s1, s2, s3,
           i0, i1, i2, i3, i4, i5, i6, i7):
    cid = lax.axis_index("c")
    sid = lax.axis_index("s")
    bufs = (b0, b1, b2, b3)
    gsem = (q0, q1, q2, q3)
    ssem = (s0, s1, s2, s3)
    isem = (i0, i1, i2, i3, i4, i5, i6, i7)

    pltpu.sync_copy(zero_hbm.at[pl.ds(sid * _RPT, _RPT)],
                    acc_sh.at[pl.ds(sid * _RPT, _RPT)])
    pltpu.sync_copy(g_hbm.at[cid, pl.ds(sid * _RPT, _RPT)],
                    src_sh.at[pl.ds(sid * _RPT, _RPT)])
    plsc.subcore_barrier()

    def idx_start(c, s):
      pltpu.async_copy(rc_hbm.at[sid, c], idx_v.at[s], isem[s])

    def idx_wait(s):
      pltpu.make_async_copy(rc_hbm.at[sid, 0], idx_v.at[s], isem[s]).wait()

    def gather_start(c, b, s, par):
      if par == 0:
        pltpu.async_copy(src_sh.at[idx_v.at[s, 0]], bufs[b], gsem[b])
      else:
        pltpu.async_copy(g_hbm.at[cid].at[idx_v.at[s, 0]], bufs[b], gsem[b])

    def gather_wait(b):
      pltpu.make_async_copy(src_sh.at[pl.ds(0, _CH)], bufs[b], gsem[b]).wait()

    def scatter_start(c, b, s):
      pltpu.async_copy(bufs[b], acc_sh.at[idx_v.at[s, 1]], ssem[b], add=True)

    def scatter_wait(b):
      pltpu.make_async_copy(zero_hbm.at[pl.ds(0, _CH)], bufs[b], ssem[b]).wait()

    _ring_pipeline(_NCHF, idx_start, idx_wait, gather_start, gather_wait,
                   scatter_start, scatter_wait)
    plsc.subcore_barrier()
    pltpu.sync_copy(acc_sh.at[pl.ds(sid * _RPT, _RPT)],
                    out_hbm.at[cid, pl.ds(sid * _RPT, _RPT)])

  return prop


def _make_prop_edge(width):
  mesh = plsc.VectorSubcoreMesh(core_axis_name="c", subcore_axis_name="s")

  @functools.partial(
      pl.kernel,
      out_type=jax.ShapeDtypeStruct((_NC, _NP, width), jnp.float32),
      mesh=mesh,
      compiler_params=_SC_PARAMS,
      scratch_types=[
          pltpu.VMEM((_NCHE, 2, _CH), jnp.int32),
          pltpu.VMEM((_CH, width), jnp.float32),
          pltpu.VMEM((_CH, width), jnp.float32),
          pltpu.VMEM((_CH, width), jnp.float32),
          pltpu.VMEM((_CH, width), jnp.float32),
          pltpu.VMEM_SHARED((_NP, width), jnp.float32),
          pltpu.VMEM_SHARED((_NP, width), jnp.float32),
          pltpu.SemaphoreType.DMA,
          pltpu.SemaphoreType.DMA,
          pltpu.SemaphoreType.DMA,
          pltpu.SemaphoreType.DMA,
          pltpu.SemaphoreType.DMA,
          pltpu.SemaphoreType.DMA,
          pltpu.SemaphoreType.DMA,
          pltpu.SemaphoreType.DMA,
      ],
  )
  def prop(g_hbm, rc_hbm, zero_hbm, out_hbm,
           idx_v, b0, b1, b2, b3, acc_sh, src_sh,
           q0, q1, q2, q3, s0, s1, s2, s3):
    cid = lax.axis_index("c")
    sid = lax.axis_index("s")
    wid = sid * _NC + cid
    bufs = (b0, b1, b2, b3)
    gsem = (q0, q1, q2, q3)
    ssem = (s0, s1, s2, s3)

    pltpu.sync_copy(zero_hbm.at[pl.ds(sid * _RPT, _RPT)],
                    acc_sh.at[pl.ds(sid * _RPT, _RPT)])
    pltpu.sync_copy(g_hbm.at[pl.ds(sid * _RPT, _RPT)],
                    src_sh.at[pl.ds(sid * _RPT, _RPT)])
    pltpu.sync_copy(rc_hbm.at[wid], idx_v)
    plsc.subcore_barrier()

    def gather_start(c, b):
      pltpu.async_copy(src_sh.at[idx_v.at[c, 0]], bufs[b], gsem[b])

    def gather_wait(b):
      pltpu.make_async_copy(src_sh.at[pl.ds(0, _CH)], bufs[b], gsem[b]).wait()

    def scatter_start(c, b):
      pltpu.async_copy(bufs[b], acc_sh.at[idx_v.at[c, 1]], ssem[b], add=True)

    def scatter_wait(b):
      pltpu.make_async_copy(zero_hbm.at[pl.ds(0, _CH)], bufs[b], ssem[b]).wait()

    _ring_pipeline_preidx(_NCHE, gather_start, gather_wait, scatter_start,
                          scatter_wait)
    plsc.subcore_barrier()
    pltpu.sync_copy(acc_sh.at[pl.ds(sid * _RPT, _RPT)],
                    out_hbm.at[cid, pl.ds(sid * _RPT, _RPT)])

  return prop


def _make_deg():
  mesh = plsc.VectorSubcoreMesh(core_axis_name="c", subcore_axis_name="s")

  @functools.partial(
      pl.kernel,
      out_type=jax.ShapeDtypeStruct((_NC, _NP, 8), jnp.float32),
      mesh=mesh,
      compiler_params=_SC_PARAMS,
      scratch_types=[
          pltpu.VMEM((_NCHE, _CH), jnp.int32),
          pltpu.VMEM((_CH, 8), jnp.float32),
          pltpu.VMEM_SHARED((_NP, 8), jnp.float32),
          pltpu.SemaphoreType.DMA,
      ],
  )
  def deg(col_hbm, ones_hbm, zero_hbm, out_hbm, col_v, ones_v, acc_sh, sem):
    cid = lax.axis_index("c")
    sid = lax.axis_index("s")
    wid = sid * _NC + cid
    pltpu.sync_copy(zero_hbm.at[pl.ds(sid * _RPT, _RPT)],
                    acc_sh.at[pl.ds(sid * _RPT, _RPT)])
    pltpu.sync_copy(col_hbm.at[wid], col_v)
    pltpu.sync_copy(ones_hbm, ones_v)
    plsc.subcore_barrier()

    def drain_one():
      pltpu.make_async_copy(ones_hbm, ones_v, sem).wait()

    def body(i, carry):
      pltpu.async_copy(ones_v, acc_sh.at[col_v.at[i]], sem, add=True)

      @pl.when(i >= 4)
      def _():
        drain_one()
      return carry

    lax.fori_loop(0, _NCHE, body, 0)
    for _ in range(4):
      drain_one()
    plsc.subcore_barrier()
    pltpu.sync_copy(acc_sh.at[pl.ds(sid * _RPT, _RPT)],
                    out_hbm.at[cid, pl.ds(sid * _RPT, _RPT)])

  return deg


def _matmul1(x_p, w1):
  def body(x_ref, w1_ref, t1_ref):
    t1_ref[...] = jnp.dot(x_ref[...], w1_ref[...],
                          preferred_element_type=jnp.float32)

  return pl.pallas_call(body, out_shape=jax.ShapeDtypeStruct(
      (_NP, 128), jnp.float32))(x_p, w1)


def _stage_first(pdeg, t1):
  def body(pdeg_ref, t1_ref, dinv_ref, g1_ref):
    deg = pdeg_ref[0, :, 0:1] + pdeg_ref[1, :, 0:1] + 1.0
    dinv = lax.rsqrt(deg)
    g1 = dinv * t1_ref[...]
    dinv_ref[...] = dinv
    g1_ref[...] = jnp.concatenate([g1[:, :64], g1[:, 64:]], axis=0)

  return pl.pallas_call(body, out_shape=(
      jax.ShapeDtypeStruct((_NP, 1), jnp.float32),
      jax.ShapeDtypeStruct((_NC * _NP, 64), jnp.float32),
  ))(pdeg, t1)


def _stage_mid1(p, t, dinv, b_row, w_next):
  wout = w_next.shape[1]

  def body(p_ref, t_ref, dinv_ref, b_ref, w_ref, tn_ref, gn_ref):
    dinv = dinv_ref[...]
    full = jnp.concatenate([p_ref[0], p_ref[1]], axis=1)
    s = dinv * full + dinv * dinv * t_ref[...] + b_ref[...]
    h = jnp.where(s >= 0.0, s, 0.01 * s)
    tn = jnp.dot(h, w_ref[...], preferred_element_type=jnp.float32)
    tn_ref[...] = tn
    gn_ref[...] = dinv * tn

  return pl.pallas_call(body, out_shape=(
      jax.ShapeDtypeStruct((_NP, wout), jnp.float32),
      jax.ShapeDtypeStruct((_NP, wout), jnp.float32),
  ))(p, t, dinv, b_row, w_next)


def _stage_mid(p, t, dinv, b_row, w_next):
  wout = w_next.shape[1]

  def body(p_ref, t_ref, dinv_ref, b_ref, w_ref, tn_ref, gn_ref):
    dinv = dinv_ref[...]
    s = dinv * (p_ref[0] + p_ref[1]) + dinv * dinv * t_ref[...] + b_ref[...]
    h = jnp.where(s >= 0.0, s, 0.01 * s)
    tn = jnp.dot(h, w_ref[...], preferred_element_type=jnp.float32)
    tn_ref[...] = tn
    gn_ref[...] = dinv * tn

  return pl.pallas_call(body, out_shape=(
      jax.ShapeDtypeStruct((_NP, wout), jnp.float32),
      jax.ShapeDtypeStruct((_NP, wout), jnp.float32),
  ))(p, t, dinv, b_row, w_next)


def _stage_final(p, t, dinv, b_row, batch_col, wlin, blin_row):
  def body(p_ref, t_ref, dinv_ref, b_ref, batch_ref, wlin_ref, blin_ref, out_ref):
    dinv = dinv_ref[...]
    s = dinv * (p_ref[0] + p_ref[1]) + dinv * dinv * t_ref[...] + b_ref[...]
    h = jnp.where(s >= 0.0, s, 0.01 * s)
    h4 = h[:, 0:1]
    gid = lax.broadcasted_iota(jnp.int32, (_NP, _G), 1)
    onehot = (batch_ref[...] == gid).astype(jnp.float32)
    sums = jnp.sum(onehot * h4, axis=0)
    counts = jnp.sum(onehot, axis=0)
    mean = sums / jnp.maximum(counts, 1.0)
    out_ref[...] = jax.nn.sigmoid(mean[:, None] * wlin_ref[...] + blin_ref[...])

  return pl.pallas_call(body, out_shape=jax.ShapeDtypeStruct((_G, 2), jnp.float32))(
      p, t, dinv, b_row, batch_col, wlin, blin_row)


def kernel(x, edge_index, batch, W1, b1, W2, b2, W3, b3, W4, b4, Wlin, blin):
  f32 = jnp.float32
  row = edge_index[0]
  col = edge_index[1]
  pad_e = _EP - _E
  row_flat = jnp.concatenate([row, jnp.zeros((pad_e,), jnp.int32)])
  col_flat = jnp.concatenate([col, jnp.full((pad_e,), _N, jnp.int32)])
  row_e = row_flat.reshape(_NW, _NCHE, _CH)
  col_e = col_flat.reshape(_NW, _NCHE, _CH)
  rc_e = jnp.stack([row_e, col_e], axis=2)
  rc_f = jnp.stack([row_flat.reshape(_NS, _NCHF, _CH),
                    col_flat.reshape(_NS, _NCHF, _CH)], axis=2)
  x_p = jnp.concatenate([x, jnp.zeros((_NP - _N, 128), f32)])
  batch_col = jnp.concatenate([batch, jnp.full((_NP - _N,), _G, jnp.int32)])[:, None]

  zeros64 = jnp.zeros((_NP, 64), f32)
  zeros16 = jnp.zeros((_NP, 16), f32)
  zeros8 = jnp.zeros((_NP, 8), f32)
  ones8 = jnp.ones((_CH, 8), f32)

  w2p = jnp.zeros((128, 16), f32).at[:, :4].set(W2)
  w3p = jnp.zeros((16, 16), f32).at[:4, :2].set(W3)
  w4p = jnp.zeros((16, 16), f32).at[:2, :1].set(W4)
  b1r = b1[None, :]
  b2r = jnp.zeros((1, 16), f32).at[0, :4].set(b2)
  b3r = jnp.zeros((1, 16), f32).at[0, :2].set(b3)
  b4r = jnp.zeros((1, 16), f32).at[0, :1].set(b4)

  prop1 = _make_prop_feat()
  prop16 = _make_prop_edge(16)

  pdeg = _make_deg()(col_e, ones8, zeros8)
  t1 = _matmul1(x_p, W1)
  dinv, g1 = _stage_first(pdeg, t1)
  p1 = prop1(g1.reshape(_NC, _NP, 64), rc_f, zeros64)
  t2, g2 = _stage_mid1(p1, t1, dinv, b1r, w2p)
  p2 = prop16(g2, rc_e, zeros16)
  t3, g3 = _stage_mid(p2, t2, dinv, b2r, w3p)
  p3 = prop16(g3, rc_e, zeros16)
  t4, g4 = _stage_mid(p3, t3, dinv, b3r, w4p)
  p4 = prop16(g4, rc_e, zeros16)
  return _stage_final(p4, t4, dinv, b4r, batch_col, Wlin, blin[None, :])


# speedup vs baseline: 1.1650x; 1.1650x over previous
import functools

import jax
import jax.numpy as jnp
from jax import lax
from jax.experimental import pallas as pl
from jax.experimental.pallas import tpu as pltpu
from jax.experimental.pallas import tpu_sc as plsc

_N = 10000
_E = 320000
_G = 64
_NP = 10240
_NC = 2
_NS = 16
_NW = _NC * _NS
_EP = 327680
_CH = 128
_NCHE = _EP // (_NW * _CH)
_NCHF = _EP // (_NS * _CH)
_RPT = _NP // _NS

_SC_PARAMS = pltpu.CompilerParams(use_tc_tiling_on_sc=False)


def _ring_pipeline(nch, idx_start, idx_wait, gather_start, gather_wait,
                   scatter_start, scatter_wait):
  for c in range(8):
    idx_start(c, c)
  for c in range(3):
    idx_wait(c)
    gather_start(c, c, c, c % 2)

  def body(j, carry):
    c0 = j * 8
    for t in range(8):
      c = c0 + t
      cn = c + 3
      ci = c + 7
      bn = (t + 3) % 4
      sn = (t + 3) % 8
      bi = (t + 7) % 8

      @pl.when(c >= 1)
      def _():
        scatter_wait((t + 3) % 4)

      @pl.when(ci < nch)
      def _():
        idx_start(ci, bi)

      @pl.when(cn < nch)
      def _():
        idx_wait(sn)
        gather_start(cn, bn, sn, (t + 3) % 2)

      gather_wait(t % 4)
      scatter_start(c, t % 4, t % 8)
    return carry

  lax.fori_loop(0, nch // 8, body, 0)
  scatter_wait((nch - 1) % 4)


def _ring_pipeline_preidx(nch, gather_start, gather_wait, scatter_start,
                          scatter_wait):
  for b in range(3):
    gather_start(b, b)

  def body(j, carry):
    c0 = j * 4
    for t in range(4):
      c = c0 + t
      cn = c + 3
      bn = (t + 3) % 4

      @pl.when(cn < nch)
      def _():
        @pl.when(c >= 1)
        def _():
          scatter_wait(bn)
        gather_start(cn, bn)

      gather_wait(t)
      scatter_start(c, t)
    return carry

  lax.fori_loop(0, nch // 4, body, 0)
  for b in range(4):
    scatter_wait(b)


def _make_prop_feat():
  mesh = plsc.VectorSubcoreMesh(core_axis_name="c", subcore_axis_name="s")

  @functools.partial(
      pl.kernel,
      out_type=jax.ShapeDtypeStruct((_NC, _NP, 64), jnp.float32),
      mesh=mesh,
      compiler_params=_SC_PARAMS,
      scratch_types=[
          pltpu.VMEM((8, 2, _CH), jnp.int32),
          pltpu.VMEM((_CH, 64), jnp.float32),
          pltpu.VMEM((_CH, 64), jnp.float32),
          pltpu.VMEM((_CH, 64), jnp.float32),
          pltpu.VMEM((_CH, 64), jnp.float32),
          pltpu.VMEM_SHARED((_NP, 64), jnp.float32),
          pltpu.VMEM_SHARED((_NP, 64), jnp.float32),
          pltpu.SemaphoreType.DMA,
          pltpu.SemaphoreType.DMA,
          pltpu.SemaphoreType.DMA,
          pltpu.SemaphoreType.DMA,
          pltpu.SemaphoreType.DMA,
          pltpu.SemaphoreType.DMA,
          pltpu.SemaphoreType.DMA,
          pltpu.SemaphoreType.DMA,
          pltpu.SemaphoreType.DMA,
          pltpu.SemaphoreType.DMA,
          pltpu.SemaphoreType.DMA,
          pltpu.SemaphoreType.DMA,
          pltpu.SemaphoreType.DMA,
          pltpu.SemaphoreType.DMA,
          pltpu.SemaphoreType.DMA,
          pltpu.SemaphoreType.DMA,
      ],
  )
  def prop(g_hbm, rc_hbm, zero_hbm, out_hbm,
           idx_v, b0, b1, b2, b3, acc_sh, src_sh,
           q0, q1, q2, q3, s0, s1, s2, s3,
           i0, i1, i2, i3, i4, i5, i6, i7):
    cid = lax.axis_index("c")
    sid = lax.axis_index("s")
    bufs = (b0, b1, b2, b3)
    gsem = (q0, q1, q2, q3)
    ssem = (s0, s1, s2, s3)
    isem = (i0, i1, i2, i3, i4, i5, i6, i7)

    pltpu.sync_copy(zero_hbm.at[pl.ds(sid * _RPT, _RPT)],
                    acc_sh.at[pl.ds(sid * _RPT, _RPT)])
    pltpu.sync_copy(g_hbm.at[cid, pl.ds(sid * _RPT, _RPT)],
                    src_sh.at[pl.ds(sid * _RPT, _RPT)])
    plsc.subcore_barrier()

    def idx_start(c, s):
      pltpu.async_copy(rc_hbm.at[sid, c], idx_v.at[s], isem[s])

    def idx_wait(s):
      pltpu.make_async_copy(rc_hbm.at[sid, 0], idx_v.at[s], isem[s]).wait()

    def gather_start(c, b, s, par):
      del par
      pltpu.async_copy(src_sh.at[idx_v.at[s, 0]], bufs[b], gsem[b])

    def gather_wait(b):
      pltpu.make_async_copy(src_sh.at[pl.ds(0, _CH)], bufs[b], gsem[b]).wait()

    def scatter_start(c, b, s):
      pltpu.async_copy(bufs[b], acc_sh.at[idx_v.at[s, 1]], ssem[b], add=True)

    def scatter_wait(b):
      pltpu.make_async_copy(zero_hbm.at[pl.ds(0, _CH)], bufs[b], ssem[b]).wait()

    _ring_pipeline(_NCHF, idx_start, idx_wait, gather_start, gather_wait,
                   scatter_start, scatter_wait)
    plsc.subcore_barrier()
    pltpu.sync_copy(acc_sh.at[pl.ds(sid * _RPT, _RPT)],
                    out_hbm.at[cid, pl.ds(sid * _RPT, _RPT)])

  return prop


def _make_prop_edge(width):
  mesh = plsc.VectorSubcoreMesh(core_axis_name="c", subcore_axis_name="s")

  @functools.partial(
      pl.kernel,
      out_type=jax.ShapeDtypeStruct((_NC, _NP, width), jnp.float32),
      mesh=mesh,
      compiler_params=_SC_PARAMS,
      scratch_types=[
          pltpu.VMEM((_NCHE, 2, _CH), jnp.int32),
          pltpu.VMEM((_CH, width), jnp.float32),
          pltpu.VMEM((_CH, width), jnp.float32),
          pltpu.VMEM((_CH, width), jnp.float32),
          pltpu.VMEM((_CH, width), jnp.float32),
          pltpu.VMEM_SHARED((_NP, width), jnp.float32),
          pltpu.VMEM_SHARED((_NP, width), jnp.float32),
          pltpu.SemaphoreType.DMA,
          pltpu.SemaphoreType.DMA,
          pltpu.SemaphoreType.DMA,
          pltpu.SemaphoreType.DMA,
          pltpu.SemaphoreType.DMA,
          pltpu.SemaphoreType.DMA,
          pltpu.SemaphoreType.DMA,
          pltpu.SemaphoreType.DMA,
      ],
  )
  def prop(g_hbm, rc_hbm, zero_hbm, out_hbm,
           idx_v, b0, b1, b2, b3, acc_sh, src_sh,
           q0, q1, q2, q3, s0, s1, s2, s3):
    cid = lax.axis_index("c")
    sid = lax.axis_index("s")
    wid = sid * _NC + cid
    bufs = (b0, b1, b2, b3)
    gsem = (q0, q1, q2, q3)
    ssem = (s0, s1, s2, s3)

    pltpu.sync_copy(zero_hbm.at[pl.ds(sid * _RPT, _RPT)],
                    acc_sh.at[pl.ds(sid * _RPT, _RPT)])
    pltpu.sync_copy(g_hbm.at[pl.ds(sid * _RPT, _RPT)],
                    src_sh.at[pl.ds(sid * _RPT, _RPT)])
    pltpu.sync_copy(rc_hbm.at[wid], idx_v)
    plsc.subcore_barrier()

    def gather_start(c, b):
      pltpu.async_copy(src_sh.at[idx_v.at[c, 0]], bufs[b], gsem[b])

    def gather_wait(b):
      pltpu.make_async_copy(src_sh.at[pl.ds(0, _CH)], bufs[b], gsem[b]).wait()

    def scatter_start(c, b):
      pltpu.async_copy(bufs[b], acc_sh.at[idx_v.at[c, 1]], ssem[b], add=True)

    def scatter_wait(b):
      pltpu.make_async_copy(zero_hbm.at[pl.ds(0, _CH)], bufs[b], ssem[b]).wait()

    _ring_pipeline_preidx(_NCHE, gather_start, gather_wait, scatter_start,
                          scatter_wait)
    plsc.subcore_barrier()
    pltpu.sync_copy(acc_sh.at[pl.ds(sid * _RPT, _RPT)],
                    out_hbm.at[cid, pl.ds(sid * _RPT, _RPT)])

  return prop


def _make_deg():
  mesh = plsc.VectorSubcoreMesh(core_axis_name="c", subcore_axis_name="s")

  @functools.partial(
      pl.kernel,
      out_type=jax.ShapeDtypeStruct((_NC, _NP, 8), jnp.float32),
      mesh=mesh,
      compiler_params=_SC_PARAMS,
      scratch_types=[
          pltpu.VMEM((_NCHE, _CH), jnp.int32),
          pltpu.VMEM((_CH, 8), jnp.float32),
          pltpu.VMEM_SHARED((_NP, 8), jnp.float32),
          pltpu.SemaphoreType.DMA,
      ],
  )
  def deg(col_hbm, ones_hbm, zero_hbm, out_hbm, col_v, ones_v, acc_sh, sem):
    cid = lax.axis_index("c")
    sid = lax.axis_index("s")
    wid = sid * _NC + cid
    pltpu.sync_copy(zero_hbm.at[pl.ds(sid * _RPT, _RPT)],
                    acc_sh.at[pl.ds(sid * _RPT, _RPT)])
    pltpu.sync_copy(col_hbm.at[wid], col_v)
    pltpu.sync_copy(ones_hbm, ones_v)
    plsc.subcore_barrier()

    def drain_one():
      pltpu.make_async_copy(ones_hbm, ones_v, sem).wait()

    def body(i, carry):
      pltpu.async_copy(ones_v, acc_sh.at[col_v.at[i]], sem, add=True)

      @pl.when(i >= 4)
      def _():
        drain_one()
      return carry

    lax.fori_loop(0, _NCHE, body, 0)
    for _ in range(4):
      drain_one()
    plsc.subcore_barrier()
    pltpu.sync_copy(acc_sh.at[pl.ds(sid * _RPT, _RPT)],
                    out_hbm.at[cid, pl.ds(sid * _RPT, _RPT)])

  return deg


def _matmul1(x_p, w1):
  def body(x_ref, w1_ref, t1_ref):
    t1_ref[...] = jnp.dot(x_ref[...], w1_ref[...],
                          preferred_element_type=jnp.float32)

  return pl.pallas_call(body, out_shape=jax.ShapeDtypeStruct(
      (_NP, 128), jnp.float32))(x_p, w1)


def _stage_first(pdeg, t1):
  def body(pdeg_ref, t1_ref, dinv_ref, g1_ref):
    deg = pdeg_ref[0, :, 0:1] + pdeg_ref[1, :, 0:1] + 1.0
    dinv = lax.rsqrt(deg)
    g1 = dinv * t1_ref[...]
    dinv_ref[...] = dinv
    g1_ref[...] = jnp.concatenate([g1[:, :64], g1[:, 64:]], axis=0)

  return pl.pallas_call(body, out_shape=(
      jax.ShapeDtypeStruct((_NP, 1), jnp.float32),
      jax.ShapeDtypeStruct((_NC * _NP, 64), jnp.float32),
  ))(pdeg, t1)


def _stage_mid1(p, t, dinv, b_row, w_next):
  wout = w_next.shape[1]

  def body(p_ref, t_ref, dinv_ref, b_ref, w_ref, tn_ref, gn_ref):
    dinv = dinv_ref[...]
    full = jnp.concatenate([p_ref[0], p_ref[1]], axis=1)
    s = dinv * full + dinv * dinv * t_ref[...] + b_ref[...]
    h = jnp.where(s >= 0.0, s, 0.01 * s)
    tn = jnp.dot(h, w_ref[...], preferred_element_type=jnp.float32)
    tn_ref[...] = tn
    gn_ref[...] = dinv * tn

  return pl.pallas_call(body, out_shape=(
      jax.ShapeDtypeStruct((_NP, wout), jnp.float32),
      jax.ShapeDtypeStruct((_NP, wout), jnp.float32),
  ))(p, t, dinv, b_row, w_next)


def _stage_mid(p, t, dinv, b_row, w_next):
  wout = w_next.shape[1]

  def body(p_ref, t_ref, dinv_ref, b_ref, w_ref, tn_ref, gn_ref):
    dinv = dinv_ref[...]
    s = dinv * (p_ref[0] + p_ref[1]) + dinv * dinv * t_ref[...] + b_ref[...]
    h = jnp.where(s >= 0.0, s, 0.01 * s)
    tn = jnp.dot(h, w_ref[...], preferred_element_type=jnp.float32)
    tn_ref[...] = tn
    gn_ref[...] = dinv * tn

  return pl.pallas_call(body, out_shape=(
      jax.ShapeDtypeStruct((_NP, wout), jnp.float32),
      jax.ShapeDtypeStruct((_NP, wout), jnp.float32),
  ))(p, t, dinv, b_row, w_next)


def _stage_final(p, t, dinv, b_row, batch_col, wlin, blin_row):
  def body(p_ref, t_ref, dinv_ref, b_ref, batch_ref, wlin_ref, blin_ref, out_ref):
    dinv = dinv_ref[...]
    s = dinv * (p_ref[0] + p_ref[1]) + dinv * dinv * t_ref[...] + b_ref[...]
    h = jnp.where(s >= 0.0, s, 0.01 * s)
    h4 = h[:, 0:1]
    gid = lax.broadcasted_iota(jnp.int32, (_NP, _G), 1)
    onehot = (batch_ref[...] == gid).astype(jnp.float32)
    sums = jnp.sum(onehot * h4, axis=0)
    counts = jnp.sum(onehot, axis=0)
    mean = sums / jnp.maximum(counts, 1.0)
    out_ref[...] = jax.nn.sigmoid(mean[:, None] * wlin_ref[...] + blin_ref[...])

  return pl.pallas_call(body, out_shape=jax.ShapeDtypeStruct((_G, 2), jnp.float32))(
      p, t, dinv, b_row, batch_col, wlin, blin_row)


def kernel(x, edge_index, batch, W1, b1, W2, b2, W3, b3, W4, b4, Wlin, blin):
  f32 = jnp.float32
  row = edge_index[0]
  col = edge_index[1]
  pad_e = _EP - _E
  row_flat = jnp.concatenate([row, jnp.zeros((pad_e,), jnp.int32)])
  col_flat = jnp.concatenate([col, jnp.full((pad_e,), _N, jnp.int32)])
  row_e = row_flat.reshape(_NW, _NCHE, _CH)
  col_e = col_flat.reshape(_NW, _NCHE, _CH)
  rc_e = jnp.stack([row_e, col_e], axis=2)
  rc_f = jnp.stack([row_flat.reshape(_NS, _NCHF, _CH),
                    col_flat.reshape(_NS, _NCHF, _CH)], axis=2)
  x_p = jnp.concatenate([x, jnp.zeros((_NP - _N, 128), f32)])
  batch_col = jnp.concatenate([batch, jnp.full((_NP - _N,), _G, jnp.int32)])[:, None]

  zeros64 = jnp.zeros((_NP, 64), f32)
  zeros16 = jnp.zeros((_NP, 16), f32)
  zeros8 = jnp.zeros((_NP, 8), f32)
  ones8 = jnp.ones((_CH, 8), f32)

  w2p = jnp.zeros((128, 16), f32).at[:, :4].set(W2)
  w3p = jnp.zeros((16, 16), f32).at[:4, :2].set(W3)
  w4p = jnp.zeros((16, 16), f32).at[:2, :1].set(W4)
  b1r = b1[None, :]
  b2r = jnp.zeros((1, 16), f32).at[0, :4].set(b2)
  b3r = jnp.zeros((1, 16), f32).at[0, :2].set(b3)
  b4r = jnp.zeros((1, 16), f32).at[0, :1].set(b4)

  prop1 = _make_prop_feat()
  prop16 = _make_prop_edge(16)

  pdeg = _make_deg()(col_e, ones8, zeros8)
  t1 = _matmul1(x_p, W1)
  dinv, g1 = _stage_first(pdeg, t1)
  p1 = prop1(g1.reshape(_NC, _NP, 64), rc_f, zeros64)
  t2, g2 = _stage_mid1(p1, t1, dinv, b1r, w2p)
  p2 = prop16(g2, rc_e, zeros16)
  t3, g3 = _stage_mid(p2, t2, dinv, b2r, w3p)
  p3 = prop16(g3, rc_e, zeros16)
  t4, g4 = _stage_mid(p3, t3, dinv, b3r, w4p)
  p4 = prop16(g4, rc_e, zeros16)
  return _stage_final(p4, t4, dinv, b4r, batch_col, Wlin, blin[None, :])


# speedup vs baseline: 1.2380x; 1.0626x over previous
import functools

import jax
import jax.numpy as jnp
from jax import lax
from jax.experimental import pallas as pl
from jax.experimental.pallas import tpu as pltpu
from jax.experimental.pallas import tpu_sc as plsc

_N = 10000
_E = 320000
_G = 64
_NP = 10240
_NC = 2
_NS = 16
_NW = _NC * _NS
_EP = 327680
_CH = 128
_NCHE = _EP // (_NW * _CH)
_NCHF = _EP // (_NS * _CH)
_RPT = _NP // _NS

_SC_PARAMS = pltpu.CompilerParams(use_tc_tiling_on_sc=False)


def _ring_pipeline(nch, idx_start, idx_wait, gather_start, gather_wait,
                   scatter_start, scatter_wait):
  for c in range(8):
    idx_start(c, c)
  for c in range(3):
    idx_wait(c)
    gather_start(c, c, c, c % 2)

  def body(j, carry):
    c0 = j * 8
    for t in range(8):
      c = c0 + t
      cn = c + 3
      ci = c + 7
      bn = (t + 3) % 4
      sn = (t + 3) % 8
      bi = (t + 7) % 8

      @pl.when(c >= 1)
      def _():
        scatter_wait((t + 3) % 4)

      @pl.when(ci < nch)
      def _():
        idx_start(ci, bi)

      @pl.when(cn < nch)
      def _():
        idx_wait(sn)
        gather_start(cn, bn, sn, (t + 3) % 2)

      gather_wait(t % 4)
      scatter_start(c, t % 4, t % 8)
    return carry

  lax.fori_loop(0, nch // 8, body, 0)
  scatter_wait((nch - 1) % 4)


def _ring_pipeline5(nch, idx_start, idx_wait, gather_start, gather_wait,
                    scatter_start, scatter_wait):
  for c in range(8):
    idx_start(c, c)
  for c in range(3):
    idx_wait(c)
    gather_start(c, c, c, 0)

  def body(j, carry):
    c0 = j * 40
    for t in range(40):
      c = c0 + t
      cn = c + 3
      ci = c + 8

      @pl.when(c >= 2)
      def _():
        scatter_wait((t + 3) % 5)

      @pl.when(ci < nch)
      def _():
        idx_start(ci, (t + 8) % 10)

      @pl.when(cn < nch)
      def _():
        idx_wait((t + 3) % 10)
        gather_start(cn, (t + 3) % 5, (t + 3) % 10, 0)

      gather_wait(t % 5)
      scatter_start(c, t % 5, t % 10)
    return carry

  lax.fori_loop(0, nch // 40, body, 0)
  scatter_wait((nch - 2) % 5)
  scatter_wait((nch - 1) % 5)


def _ring_pipeline_preidx(nch, gather_start, gather_wait, scatter_start,
                          scatter_wait):
  for b in range(3):
    gather_start(b, b)

  def body(j, carry):
    c0 = j * 4
    for t in range(4):
      c = c0 + t
      cn = c + 3
      bn = (t + 3) % 4

      @pl.when(cn < nch)
      def _():
        @pl.when(c >= 1)
        def _():
          scatter_wait(bn)
        gather_start(cn, bn)

      gather_wait(t)
      scatter_start(c, t)
    return carry

  lax.fori_loop(0, nch // 4, body, 0)
  for b in range(4):
    scatter_wait(b)


def _make_prop_feat():
  mesh = plsc.VectorSubcoreMesh(core_axis_name="c", subcore_axis_name="s")

  @functools.partial(
      pl.kernel,
      out_type=jax.ShapeDtypeStruct((_NC, _NP, 64), jnp.float32),
      mesh=mesh,
      compiler_params=_SC_PARAMS,
      scratch_types=[
          pltpu.VMEM((10, 2, _CH), jnp.int32),
          pltpu.VMEM((_CH, 64), jnp.float32),
          pltpu.VMEM((_CH, 64), jnp.float32),
          pltpu.VMEM((_CH, 64), jnp.float32),
          pltpu.VMEM((_CH, 64), jnp.float32),
          pltpu.VMEM((_CH, 64), jnp.float32),
          pltpu.VMEM_SHARED((_NP, 64), jnp.float32),
          pltpu.VMEM_SHARED((_NP, 64), jnp.float32),
      ] + [pltpu.SemaphoreType.DMA] * 20,
  )
  def prop(g_hbm, rc_hbm, zero_hbm, out_hbm,
           idx_v, b0, b1, b2, b3, b4, acc_sh, src_sh,
           q0, q1, q2, q3, q4, s0, s1, s2, s3, s4,
           i0, i1, i2, i3, i4, i5, i6, i7, i8, i9):
    cid = lax.axis_index("c")
    sid = lax.axis_index("s")
    bufs = (b0, b1, b2, b3, b4)
    gsem = (q0, q1, q2, q3, q4)
    ssem = (s0, s1, s2, s3, s4)
    isem = (i0, i1, i2, i3, i4, i5, i6, i7, i8, i9)

    pltpu.sync_copy(zero_hbm.at[pl.ds(sid * _RPT, _RPT)],
                    acc_sh.at[pl.ds(sid * _RPT, _RPT)])
    pltpu.sync_copy(g_hbm.at[cid, pl.ds(sid * _RPT, _RPT)],
                    src_sh.at[pl.ds(sid * _RPT, _RPT)])
    plsc.subcore_barrier()

    def idx_start(c, s):
      pltpu.async_copy(rc_hbm.at[sid, c], idx_v.at[s], isem[s])

    def idx_wait(s):
      pltpu.make_async_copy(rc_hbm.at[sid, 0], idx_v.at[s], isem[s]).wait()

    def gather_start(c, b, s, par):
      del par
      pltpu.async_copy(src_sh.at[idx_v.at[s, 0]], bufs[b], gsem[b])

    def gather_wait(b):
      pltpu.make_async_copy(src_sh.at[pl.ds(0, _CH)], bufs[b], gsem[b]).wait()

    def scatter_start(c, b, s):
      pltpu.async_copy(bufs[b], acc_sh.at[idx_v.at[s, 1]], ssem[b], add=True)

    def scatter_wait(b):
      pltpu.make_async_copy(zero_hbm.at[pl.ds(0, _CH)], bufs[b], ssem[b]).wait()

    _ring_pipeline5(_NCHF, idx_start, idx_wait, gather_start, gather_wait,
                    scatter_start, scatter_wait)
    plsc.subcore_barrier()
    pltpu.sync_copy(acc_sh.at[pl.ds(sid * _RPT, _RPT)],
                    out_hbm.at[cid, pl.ds(sid * _RPT, _RPT)])

  return prop


def _make_prop_edge(width):
  mesh = plsc.VectorSubcoreMesh(core_axis_name="c", subcore_axis_name="s")

  @functools.partial(
      pl.kernel,
      out_type=jax.ShapeDtypeStruct((_NC, _NP, width), jnp.float32),
      mesh=mesh,
      compiler_params=_SC_PARAMS,
      scratch_types=[
          pltpu.VMEM((_NCHE, 2, _CH), jnp.int32),
          pltpu.VMEM((_CH, width), jnp.float32),
          pltpu.VMEM((_CH, width), jnp.float32),
          pltpu.VMEM((_CH, width), jnp.float32),
          pltpu.VMEM((_CH, width), jnp.float32),
          pltpu.VMEM_SHARED((_NP, width), jnp.float32),
          pltpu.VMEM_SHARED((_NP, width), jnp.float32),
          pltpu.SemaphoreType.DMA,
          pltpu.SemaphoreType.DMA,
          pltpu.SemaphoreType.DMA,
          pltpu.SemaphoreType.DMA,
          pltpu.SemaphoreType.DMA,
          pltpu.SemaphoreType.DMA,
          pltpu.SemaphoreType.DMA,
          pltpu.SemaphoreType.DMA,
      ],
  )
  def prop(g_hbm, rc_hbm, zero_hbm, out_hbm,
           idx_v, b0, b1, b2, b3, acc_sh, src_sh,
           q0, q1, q2, q3, s0, s1, s2, s3):
    cid = lax.axis_index("c")
    sid = lax.axis_index("s")
    wid = sid * _NC + cid
    bufs = (b0, b1, b2, b3)
    gsem = (q0, q1, q2, q3)
    ssem = (s0, s1, s2, s3)

    pltpu.sync_copy(zero_hbm.at[pl.ds(sid * _RPT, _RPT)],
                    acc_sh.at[pl.ds(sid * _RPT, _RPT)])
    pltpu.sync_copy(g_hbm.at[pl.ds(sid * _RPT, _RPT)],
                    src_sh.at[pl.ds(sid * _RPT, _RPT)])
    pltpu.sync_copy(rc_hbm.at[wid], idx_v)
    plsc.subcore_barrier()

    def gather_start(c, b):
      pltpu.async_copy(src_sh.at[idx_v.at[c, 0]], bufs[b], gsem[b])

    def gather_wait(b):
      pltpu.make_async_copy(src_sh.at[pl.ds(0, _CH)], bufs[b], gsem[b]).wait()

    def scatter_start(c, b):
      pltpu.async_copy(bufs[b], acc_sh.at[idx_v.at[c, 1]], ssem[b], add=True)

    def scatter_wait(b):
      pltpu.make_async_copy(zero_hbm.at[pl.ds(0, _CH)], bufs[b], ssem[b]).wait()

    _ring_pipeline_preidx(_NCHE, gather_start, gather_wait, scatter_start,
                          scatter_wait)
    plsc.subcore_barrier()
    pltpu.sync_copy(acc_sh.at[pl.ds(sid * _RPT, _RPT)],
                    out_hbm.at[cid, pl.ds(sid * _RPT, _RPT)])

  return prop


def _make_deg():
  mesh = plsc.VectorSubcoreMesh(core_axis_name="c", subcore_axis_name="s")

  @functools.partial(
      pl.kernel,
      out_type=jax.ShapeDtypeStruct((_NC, _NP, 8), jnp.float32),
      mesh=mesh,
      compiler_params=_SC_PARAMS,
      scratch_types=[
          pltpu.VMEM((_NCHE, _CH), jnp.int32),
          pltpu.VMEM((_CH, 8), jnp.float32),
          pltpu.VMEM_SHARED((_NP, 8), jnp.float32),
          pltpu.SemaphoreType.DMA,
      ],
  )
  def deg(col_hbm, ones_hbm, zero_hbm, out_hbm, col_v, ones_v, acc_sh, sem):
    cid = lax.axis_index("c")
    sid = lax.axis_index("s")
    wid = sid * _NC + cid
    pltpu.sync_copy(zero_hbm.at[pl.ds(sid * _RPT, _RPT)],
                    acc_sh.at[pl.ds(sid * _RPT, _RPT)])
    pltpu.sync_copy(col_hbm.at[wid], col_v)
    pltpu.sync_copy(ones_hbm, ones_v)
    plsc.subcore_barrier()

    def drain_one():
      pltpu.make_async_copy(ones_hbm, ones_v, sem).wait()

    def body(i, carry):
      pltpu.async_copy(ones_v, acc_sh.at[col_v.at[i]], sem, add=True)

      @pl.when(i >= 4)
      def _():
        drain_one()
      return carry

    lax.fori_loop(0, _NCHE, body, 0)
    for _ in range(4):
      drain_one()
    plsc.subcore_barrier()
    pltpu.sync_copy(acc_sh.at[pl.ds(sid * _RPT, _RPT)],
                    out_hbm.at[cid, pl.ds(sid * _RPT, _RPT)])

  return deg


def _matmul1(x_p, w1):
  def body(x_ref, w1_ref, t1_ref):
    t1_ref[...] = jnp.dot(x_ref[...], w1_ref[...],
                          preferred_element_type=jnp.float32)

  return pl.pallas_call(body, out_shape=jax.ShapeDtypeStruct(
      (_NP, 128), jnp.float32))(x_p, w1)


def _stage_first(pdeg, t1):
  def body(pdeg_ref, t1_ref, dinv_ref, g1_ref):
    deg = pdeg_ref[0, :, 0:1] + pdeg_ref[1, :, 0:1] + 1.0
    dinv = lax.rsqrt(deg)
    g1 = dinv * t1_ref[...]
    dinv_ref[...] = dinv
    g1_ref[...] = jnp.concatenate([g1[:, :64], g1[:, 64:]], axis=0)

  return pl.pallas_call(body, out_shape=(
      jax.ShapeDtypeStruct((_NP, 1), jnp.float32),
      jax.ShapeDtypeStruct((_NC * _NP, 64), jnp.float32),
  ))(pdeg, t1)


def _stage_mid1(p, t, dinv, b_row, w_next):
  wout = w_next.shape[1]

  def body(p_ref, t_ref, dinv_ref, b_ref, w_ref, tn_ref, gn_ref):
    dinv = dinv_ref[...]
    full = jnp.concatenate([p_ref[0], p_ref[1]], axis=1)
    s = dinv * full + dinv * dinv * t_ref[...] + b_ref[...]
    h = jnp.where(s >= 0.0, s, 0.01 * s)
    tn = jnp.dot(h, w_ref[...], preferred_element_type=jnp.float32)
    tn_ref[...] = tn
    gn_ref[...] = dinv * tn

  return pl.pallas_call(body, out_shape=(
      jax.ShapeDtypeStruct((_NP, wout), jnp.float32),
      jax.ShapeDtypeStruct((_NP, wout), jnp.float32),
  ))(p, t, dinv, b_row, w_next)


def _stage_mid(p, t, dinv, b_row, w_next):
  wout = w_next.shape[1]

  def body(p_ref, t_ref, dinv_ref, b_ref, w_ref, tn_ref, gn_ref):
    dinv = dinv_ref[...]
    s = dinv * (p_ref[0] + p_ref[1]) + dinv * dinv * t_ref[...] + b_ref[...]
    h = jnp.where(s >= 0.0, s, 0.01 * s)
    tn = jnp.dot(h, w_ref[...], preferred_element_type=jnp.float32)
    tn_ref[...] = tn
    gn_ref[...] = dinv * tn

  return pl.pallas_call(body, out_shape=(
      jax.ShapeDtypeStruct((_NP, wout), jnp.float32),
      jax.ShapeDtypeStruct((_NP, wout), jnp.float32),
  ))(p, t, dinv, b_row, w_next)


def _stage_final(p, t, dinv, b_row, batch_col, wlin, blin_row):
  def body(p_ref, t_ref, dinv_ref, b_ref, batch_ref, wlin_ref, blin_ref, out_ref):
    dinv = dinv_ref[...]
    s = dinv * (p_ref[0] + p_ref[1]) + dinv * dinv * t_ref[...] + b_ref[...]
    h = jnp.where(s >= 0.0, s, 0.01 * s)
    h4 = h[:, 0:1]
    gid = lax.broadcasted_iota(jnp.int32, (_NP, _G), 1)
    onehot = (batch_ref[...] == gid).astype(jnp.float32)
    sums = jnp.sum(onehot * h4, axis=0)
    counts = jnp.sum(onehot, axis=0)
    mean = sums / jnp.maximum(counts, 1.0)
    out_ref[...] = jax.nn.sigmoid(mean[:, None] * wlin_ref[...] + blin_ref[...])

  return pl.pallas_call(body, out_shape=jax.ShapeDtypeStruct((_G, 2), jnp.float32))(
      p, t, dinv, b_row, batch_col, wlin, blin_row)


def kernel(x, edge_index, batch, W1, b1, W2, b2, W3, b3, W4, b4, Wlin, blin):
  f32 = jnp.float32
  row = edge_index[0]
  col = edge_index[1]
  pad_e = _EP - _E
  row_flat = jnp.concatenate([row, jnp.zeros((pad_e,), jnp.int32)])
  col_flat = jnp.concatenate([col, jnp.full((pad_e,), _N, jnp.int32)])
  row_e = row_flat.reshape(_NW, _NCHE, _CH)
  col_e = col_flat.reshape(_NW, _NCHE, _CH)
  rc_e = jnp.stack([row_e, col_e], axis=2)
  rc_f = jnp.stack([row_flat.reshape(_NS, _NCHF, _CH),
                    col_flat.reshape(_NS, _NCHF, _CH)], axis=2)
  x_p = jnp.concatenate([x, jnp.zeros((_NP - _N, 128), f32)])
  batch_col = jnp.concatenate([batch, jnp.full((_NP - _N,), _G, jnp.int32)])[:, None]

  zeros64 = jnp.zeros((_NP, 64), f32)
  zeros16 = jnp.zeros((_NP, 16), f32)
  zeros8 = jnp.zeros((_NP, 8), f32)
  ones8 = jnp.ones((_CH, 8), f32)

  w2p = jnp.zeros((128, 16), f32).at[:, :4].set(W2)
  w3p = jnp.zeros((16, 16), f32).at[:4, :2].set(W3)
  w4p = jnp.zeros((16, 16), f32).at[:2, :1].set(W4)
  b1r = b1[None, :]
  b2r = jnp.zeros((1, 16), f32).at[0, :4].set(b2)
  b3r = jnp.zeros((1, 16), f32).at[0, :2].set(b3)
  b4r = jnp.zeros((1, 16), f32).at[0, :1].set(b4)

  prop1 = _make_prop_feat()
  prop16 = _make_prop_edge(16)

  pdeg = _make_deg()(col_e, ones8, zeros8)
  t1 = _matmul1(x_p, W1)
  dinv, g1 = _stage_first(pdeg, t1)
  p1 = prop1(g1.reshape(_NC, _NP, 64), rc_f, zeros64)
  t2, g2 = _stage_mid1(p1, t1, dinv, b1r, w2p)
  p2 = prop16(g2, rc_e, zeros16)
  t3, g3 = _stage_mid(p2, t2, dinv, b2r, w3p)
  p3 = prop16(g3, rc_e, zeros16)
  t4, g4 = _stage_mid(p3, t3, dinv, b3r, w4p)
  p4 = prop16(g4, rc_e, zeros16)
  return _stage_final(p4, t4, dinv, b4r, batch_col, Wlin, blin[None, :])
